# baseline (speedup 1.0000x reference)
import functools

import jax
import jax.numpy as jnp
from jax import lax
from jax.experimental import pallas as pl
from jax.experimental.pallas import tpu as pltpu
from jax.experimental.pallas import tpu_sc as plsc

EMB = 1024
NE = 8
TOPK = 2
HID = 2048
GU = 2 * HID

TILE = 256
NUM_TILES = 24
PAD_ROWS = NUM_TILES * TILE
NW = 32


def _router_body(x_ref, rw_ref, rb_ref, idx_ref, w_ref):
    s = lax.dot_general(x_ref[...], rw_ref[...], (((1,), (1,)), ((), ())),
                        preferred_element_type=jnp.float32)
    s = s + rb_ref[...]
    t = s.shape[0]
    col = lax.broadcasted_iota(jnp.int32, (t, NE), 1)
    m1 = jnp.max(s, axis=1, keepdims=True)
    a1 = jnp.min(jnp.where(s == m1, col, NE), axis=1, keepdims=True)
    s2 = jnp.where(col == a1, -jnp.inf, s)
    m2 = jnp.max(s2, axis=1, keepdims=True)
    a2 = jnp.min(jnp.where(s2 == m2, col, NE), axis=1, keepdims=True)
    e2 = jnp.exp(m2 - m1)
    w1 = 1.0 / (1.0 + e2)
    idx_ref[...] = jnp.concatenate([a1, a2], axis=1)
    w_ref[...] = jnp.concatenate([w1, 1.0 - w1], axis=1)


def _route(hidden, router_w, router_b):
    t = hidden.shape[0]
    return pl.pallas_call(
        _router_body,
        out_shape=(jax.ShapeDtypeStruct((t, TOPK), jnp.int32),
                   jax.ShapeDtypeStruct((t, TOPK), jnp.float32)),
    )(hidden, router_w, router_b.reshape(1, NE))


def _sc_gather_body(hidden, row_token, out, idx_v, rows_v, sem):
    wid = lax.axis_index("s") * 2 + lax.axis_index("c")
    per_w = PAD_ROWS // NW
    ch = per_w // 2
    for j in range(2):
        base = wid * per_w + j * ch
        pltpu.sync_copy(row_token.at[pl.ds(base, ch)], idx_v)
        pltpu.async_copy(hidden.at[idx_v], rows_v, sem).wait()
        pltpu.sync_copy(rows_v, out.at[pl.ds(base, ch)])


def _sc_gather(hidden, row_token):
    ch = PAD_ROWS // NW // 2
    return pl.kernel(
        _sc_gather_body,
        mesh=plsc.VectorSubcoreMesh(core_axis_name="c", subcore_axis_name="s"),
        out_type=jax.ShapeDtypeStruct((PAD_ROWS, EMB), jnp.float32),
        scratch_types=[
            pltpu.VMEM((ch,), jnp.int32),
            pltpu.VMEM((ch, EMB), jnp.float32),
            pltpu.SemaphoreType.DMA,
        ],
    )(hidden, row_token)


def _ffn_body(e_sref, a_sref, xg_ref, guw_hbm, gb_ref, ub_ref,
              dnw_hbm, dnb_ref, w_ref, eo_ref, wgu_v, dnw_v, sem1, sem2):
    g = pl.program_id(0)

    @pl.when(a_sref[g] == 1)
    def _():
        cur = e_sref[g]
        prev = e_sref[jnp.maximum(g - 1, 0)]

        @pl.when((g == 0) | (cur != prev))
        def _():
            c1 = pltpu.make_async_copy(guw_hbm.at[cur], wgu_v, sem1)
            c2 = pltpu.make_async_copy(dnw_hbm.at[cur], dnw_v, sem2)
            c1.start()
            c2.start()
            c1.wait()
            c2.wait()

        x = xg_ref[...].astype(jnp.bfloat16)
        wg = wgu_v[:, :EMB]
        wu = wgu_v[:, EMB:]
        gate = lax.dot_general(x, wg, (((1,), (1,)), ((), ())),
                               preferred_element_type=jnp.float32) + gb_ref[0, 0]
        up = lax.dot_general(x, wu, (((1,), (1,)), ((), ())),
                             preferred_element_type=jnp.float32) + ub_ref[0, 0]
        gate = jnp.clip(gate, -7.0, 7.0)
        act = gate * jax.nn.sigmoid(1.702 * gate) * (jnp.clip(up, -7.0, 7.0) + 1.0)
        part = lax.dot_general(act.astype(jnp.bfloat16), dnw_v[...],
                               (((1,), (1,)), ((), ())),
                               preferred_element_type=jnp.float32)
        eo_ref[...] = (part + dnb_ref[0]) * w_ref[0]


def _ffn(xg, gate_up_w, gate_up_b, down_w, down_b, row_weight, tile_expert,
         tile_active):
    guw3 = gate_up_w.reshape(NE, HID, 2 * EMB).astype(jnp.bfloat16)
    dnw = down_w.astype(jnp.bfloat16)
    gb = gate_up_b[:, 0::2].reshape(NE, 1, HID)
    ub = gate_up_b[:, 1::2].reshape(NE, 1, HID)
    dnb = down_b.reshape(NE, 1, EMB)
    rw = row_weight.reshape(NUM_TILES, TILE, 1)
    grid_spec = pltpu.PrefetchScalarGridSpec(
        num_scalar_prefetch=2,
        grid=(NUM_TILES,),
        in_specs=[
            pl.BlockSpec((TILE, EMB), lambda g, e, a: (g, 0)),
            pl.BlockSpec(memory_space=pl.ANY),
            pl.BlockSpec((1, 1, HID), lambda g, e, a: (e[g], 0, 0)),
            pl.BlockSpec((1, 1, HID), lambda g, e, a: (e[g], 0, 0)),
            pl.BlockSpec(memory_space=pl.ANY),
            pl.BlockSpec((1, 1, EMB), lambda g, e, a: (e[g], 0, 0)),
            pl.BlockSpec((1, TILE, 1), lambda g, e, a: (g, 0, 0)),
        ],
        out_specs=pl.BlockSpec((TILE, EMB), lambda g, e, a: (g, 0)),
        scratch_shapes=[
            pltpu.VMEM((HID, 2 * EMB), jnp.bfloat16),
            pltpu.VMEM((EMB, HID), jnp.bfloat16),
            pltpu.SemaphoreType.DMA,
            pltpu.SemaphoreType.DMA,
        ],
    )
    return pl.pallas_call(
        _ffn_body,
        grid_spec=grid_spec,
        out_shape=jax.ShapeDtypeStruct((PAD_ROWS, EMB), jnp.float32),
    )(tile_expert, tile_active, xg, guw3, gb, ub, dnw, dnb, rw)


def _sc_combine_body(eo, pos1, pos2, out, idx1_v, idx2_v, r1_v, r2_v, sem):
    wid = lax.axis_index("s") * 2 + lax.axis_index("c")
    t = out.shape[0]
    per_w = t // NW
    ch = per_w // 2
    for j in range(2):
        base = wid * per_w + j * ch
        pltpu.sync_copy(pos1.at[pl.ds(base, ch)], idx1_v)
        pltpu.sync_copy(pos2.at[pl.ds(base, ch)], idx2_v)
        pltpu.async_copy(eo.at[idx1_v], r1_v, sem).wait()
        pltpu.async_copy(eo.at[idx2_v], r2_v, sem).wait()

        def _row(r, carry):
            for cc in range(EMB // 16):
                sl = pl.ds(cc * 16, 16)
                r1_v[r, sl] = r1_v[r, sl] + r2_v[r, sl]
            return carry

        lax.fori_loop(0, ch, _row, 0)
        pltpu.sync_copy(r1_v, out.at[pl.ds(base, ch)])


def _sc_combine(eo, pos1, pos2, t):
    ch = t // NW // 2
    return pl.kernel(
        _sc_combine_body,
        mesh=plsc.VectorSubcoreMesh(core_axis_name="c", subcore_axis_name="s"),
        out_type=jax.ShapeDtypeStruct((t, EMB), jnp.float32),
        scratch_types=[
            pltpu.VMEM((ch,), jnp.int32),
            pltpu.VMEM((ch,), jnp.int32),
            pltpu.VMEM((ch, EMB), jnp.float32),
            pltpu.VMEM((ch, EMB), jnp.float32),
            pltpu.SemaphoreType.DMA,
        ],
    )(eo, pos1, pos2)


def kernel(x, router_w, router_b, gate_up_w, gate_up_b, down_w, down_b):
    batch, seq, _ = x.shape
    hidden = x.reshape(-1, EMB)
    t = hidden.shape[0]

    idx, wts = _route(hidden, router_w, router_b)

    e_flat = idx.reshape(-1)
    onehot = (e_flat[:, None] == jnp.arange(NE, dtype=jnp.int32)[None, :])
    csum = jnp.cumsum(onehot.astype(jnp.int32), axis=0)
    counts = csum[-1]
    rank = jnp.take_along_axis(csum, e_flat[:, None], axis=1)[:, 0] - 1
    padded = ((counts + TILE - 1) // TILE) * TILE
    pstart = jnp.concatenate([jnp.zeros(1, jnp.int32), jnp.cumsum(padded)])
    total = pstart[NE]
    slot = pstart[e_flat] + rank

    tok = jnp.arange(t * TOPK, dtype=jnp.int32) // TOPK
    w_bits = lax.bitcast_convert_type(wts.reshape(-1), jnp.int32)
    packed = jnp.stack([tok, w_bits], axis=1)
    buf = jnp.zeros((PAD_ROWS, 2), jnp.int32).at[slot].set(
        packed, unique_indices=True)
    row_token = buf[:, 0]
    row_weight = lax.bitcast_convert_type(buf[:, 1], jnp.float32)

    g_starts = jnp.arange(NUM_TILES, dtype=jnp.int32) * TILE
    probe = jnp.minimum(g_starts, total - 1)
    tile_expert = jnp.searchsorted(pstart[1:], probe, side="right").astype(jnp.int32)
    tile_active = (g_starts < total).astype(jnp.int32)

    pos = slot.reshape(t, TOPK)
    pos1 = pos[:, 0]
    pos2 = pos[:, 1]

    xg = _sc_gather(hidden, row_token)
    eo = _ffn(xg, gate_up_w, gate_up_b, down_w, down_b, row_weight,
              tile_expert, tile_active)
    out = _sc_combine(eo, pos1, pos2, t)
    return out.reshape(batch, seq, EMB)


# speedup vs baseline: 1.1020x; 1.1020x over previous
import functools

import jax
import jax.numpy as jnp
from jax import lax
from jax.experimental import pallas as pl
from jax.experimental.pallas import tpu as pltpu
from jax.experimental.pallas import tpu_sc as plsc

EMB = 1024
NE = 8
TOPK = 2
HID = 2048
GU = 2 * HID

TILE = 256
NUM_TILES = 24
PAD_ROWS = NUM_TILES * TILE
NW = 32


def _router_body(x_ref, rw_ref, rb_ref, idx_ref, w_ref):
    s = lax.dot_general(x_ref[...], rw_ref[...], (((1,), (1,)), ((), ())),
                        preferred_element_type=jnp.float32)
    s = s + rb_ref[...]
    t = s.shape[0]
    col = lax.broadcasted_iota(jnp.int32, (t, NE), 1)
    m1 = jnp.max(s, axis=1, keepdims=True)
    a1 = jnp.min(jnp.where(s == m1, col, NE), axis=1, keepdims=True)
    s2 = jnp.where(col == a1, -jnp.inf, s)
    m2 = jnp.max(s2, axis=1, keepdims=True)
    a2 = jnp.min(jnp.where(s2 == m2, col, NE), axis=1, keepdims=True)
    e2 = jnp.exp(m2 - m1)
    w1 = 1.0 / (1.0 + e2)
    idx_ref[...] = jnp.concatenate([a1, a2], axis=1)
    w_ref[...] = jnp.concatenate([w1, 1.0 - w1], axis=1)


def _route(hidden, router_w, router_b):
    t = hidden.shape[0]
    return pl.pallas_call(
        _router_body,
        out_shape=(jax.ShapeDtypeStruct((t, TOPK), jnp.int32),
                   jax.ShapeDtypeStruct((t, TOPK), jnp.float32)),
    )(hidden, router_w, router_b.reshape(1, NE))


def _sc_gather_body(hidden, row_token, out, idx_v, rows_v, sem):
    wid = lax.axis_index("s") * 2 + lax.axis_index("c")
    per_w = PAD_ROWS // NW
    ch = per_w // 2
    for j in range(2):
        base = wid * per_w + j * ch
        pltpu.sync_copy(row_token.at[pl.ds(base, ch)], idx_v)
        pltpu.async_copy(hidden.at[idx_v], rows_v, sem).wait()
        pltpu.sync_copy(rows_v, out.at[pl.ds(base, ch)])


def _sc_gather(hidden, row_token):
    ch = PAD_ROWS // NW // 2
    return pl.kernel(
        _sc_gather_body,
        mesh=plsc.VectorSubcoreMesh(core_axis_name="c", subcore_axis_name="s"),
        out_type=jax.ShapeDtypeStruct((PAD_ROWS, EMB), jnp.float32),
        scratch_types=[
            pltpu.VMEM((ch,), jnp.int32),
            pltpu.VMEM((ch, EMB), jnp.float32),
            pltpu.SemaphoreType.DMA,
        ],
    )(hidden, row_token)


def _ffn_body(e_sref, a_sref, xg_ref, guw_hbm, gb_ref, ub_ref, dnw_hbm,
              dnb_ref, w_ref, eo_ref, guw_v, dn_v, guw_bf, dn_bf, sem1, sem2):
    g = pl.program_id(0)

    @pl.when(a_sref[g] == 1)
    def _():
        cur = e_sref[g]
        prev = e_sref[jnp.maximum(g - 1, 0)]

        @pl.when((g == 0) | (cur != prev))
        def _():
            c1 = pltpu.make_async_copy(guw_hbm.at[cur], guw_v, sem1)
            c2 = pltpu.make_async_copy(dnw_hbm.at[cur], dn_v, sem2)
            c1.start()
            c2.start()
            c1.wait()
            c2.wait()
            guw_bf[...] = guw_v[...].astype(jnp.bfloat16)
            dn_bf[...] = dn_v[...].astype(jnp.bfloat16)

        x = xg_ref[...].astype(jnp.bfloat16)
        gate = lax.dot_general(x, guw_bf[:, :EMB], (((1,), (1,)), ((), ())),
                               preferred_element_type=jnp.float32) + gb_ref[0, 0]
        up = lax.dot_general(x, guw_bf[:, EMB:], (((1,), (1,)), ((), ())),
                             preferred_element_type=jnp.float32) + ub_ref[0, 0]
        act = (jnp.clip(gate, -7.0, 7.0) * jax.nn.sigmoid(1.702 * gate)
               * (jnp.clip(up, -7.0, 7.0) + 1.0))
        part = lax.dot_general(act.astype(jnp.bfloat16), dn_bf[...],
                               (((1,), (1,)), ((), ())),
                               preferred_element_type=jnp.float32)
        eo_ref[...] = (part + dnb_ref[0]) * w_ref[0]


def _ffn(xg, gate_up_w, gate_up_b, down_w, down_b, row_weight, tile_expert,
         tile_active):
    guw3 = gate_up_w.reshape(NE, HID, 2 * EMB)
    gb = gate_up_b[:, 0::2].reshape(NE, 1, HID)
    ub = gate_up_b[:, 1::2].reshape(NE, 1, HID)
    dnb = down_b.reshape(NE, 1, EMB)
    rw = row_weight.reshape(NUM_TILES, TILE, 1)
    grid_spec = pltpu.PrefetchScalarGridSpec(
        num_scalar_prefetch=2,
        grid=(NUM_TILES,),
        in_specs=[
            pl.BlockSpec((TILE, EMB), lambda g, e, a: (g, 0)),
            pl.BlockSpec(memory_space=pl.ANY),
            pl.BlockSpec((1, 1, HID), lambda g, e, a: (e[g], 0, 0)),
            pl.BlockSpec((1, 1, HID), lambda g, e, a: (e[g], 0, 0)),
            pl.BlockSpec(memory_space=pl.ANY),
            pl.BlockSpec((1, 1, EMB), lambda g, e, a: (e[g], 0, 0)),
            pl.BlockSpec((1, TILE, 1), lambda g, e, a: (g, 0, 0)),
        ],
        out_specs=pl.BlockSpec((TILE, EMB), lambda g, e, a: (g, 0)),
        scratch_shapes=[
            pltpu.VMEM((HID, 2 * EMB), jnp.float32),
            pltpu.VMEM((EMB, HID), jnp.float32),
            pltpu.VMEM((HID, 2 * EMB), jnp.bfloat16),
            pltpu.VMEM((EMB, HID), jnp.bfloat16),
            pltpu.SemaphoreType.DMA,
            pltpu.SemaphoreType.DMA,
        ],
    )
    return pl.pallas_call(
        _ffn_body,
        grid_spec=grid_spec,
        out_shape=jax.ShapeDtypeStruct((PAD_ROWS, EMB), jnp.float32),
    )(tile_expert, tile_active, xg, guw3, gb, ub, down_w, dnb, rw)


def _sc_combine_body(eo, pos1, pos2, out, idx1_v, idx2_v, r1_v, r2_v, sem):
    wid = lax.axis_index("s") * 2 + lax.axis_index("c")
    t = out.shape[0]
    per_w = t // NW
    ch = per_w // 2
    for j in range(2):
        base = wid * per_w + j * ch
        pltpu.sync_copy(pos1.at[pl.ds(base, ch)], idx1_v)
        pltpu.sync_copy(pos2.at[pl.ds(base, ch)], idx2_v)
        pltpu.async_copy(eo.at[idx1_v], r1_v, sem).wait()
        pltpu.async_copy(eo.at[idx2_v], r2_v, sem).wait()

        def _row(r, carry):
            for cc in range(EMB // 16):
                sl = pl.ds(cc * 16, 16)
                r1_v[r, sl] = r1_v[r, sl] + r2_v[r, sl]
            return carry

        lax.fori_loop(0, ch, _row, 0)
        pltpu.sync_copy(r1_v, out.at[pl.ds(base, ch)])


def _sc_combine(eo, pos1, pos2, t):
    ch = t // NW // 2
    return pl.kernel(
        _sc_combine_body,
        mesh=plsc.VectorSubcoreMesh(core_axis_name="c", subcore_axis_name="s"),
        out_type=jax.ShapeDtypeStruct((t, EMB), jnp.float32),
        scratch_types=[
            pltpu.VMEM((ch,), jnp.int32),
            pltpu.VMEM((ch,), jnp.int32),
            pltpu.VMEM((ch, EMB), jnp.float32),
            pltpu.VMEM((ch, EMB), jnp.float32),
            pltpu.SemaphoreType.DMA,
        ],
    )(eo, pos1, pos2)


def kernel(x, router_w, router_b, gate_up_w, gate_up_b, down_w, down_b):
    batch, seq, _ = x.shape
    hidden = x.reshape(-1, EMB)
    t = hidden.shape[0]

    idx, wts = _route(hidden, router_w, router_b)

    e_flat = idx.reshape(-1)
    onehot = (e_flat[:, None] == jnp.arange(NE, dtype=jnp.int32)[None, :])
    csum = jnp.cumsum(onehot.astype(jnp.int32), axis=0)
    counts = csum[-1]
    rank = jnp.take_along_axis(csum, e_flat[:, None], axis=1)[:, 0] - 1
    padded = ((counts + TILE - 1) // TILE) * TILE
    pstart = jnp.concatenate([jnp.zeros(1, jnp.int32), jnp.cumsum(padded)])
    total = pstart[NE]
    slot = pstart[e_flat] + rank

    tok = jnp.arange(t * TOPK, dtype=jnp.int32) // TOPK
    w_bits = lax.bitcast_convert_type(wts.reshape(-1), jnp.int32)
    packed = jnp.stack([tok, w_bits], axis=1)
    buf = jnp.zeros((PAD_ROWS, 2), jnp.int32).at[slot].set(
        packed, unique_indices=True)
    row_token = buf[:, 0]
    row_weight = lax.bitcast_convert_type(buf[:, 1], jnp.float32)

    g_starts = jnp.arange(NUM_TILES, dtype=jnp.int32) * TILE
    probe = jnp.minimum(g_starts, total - 1)
    tile_expert = jnp.searchsorted(pstart[1:], probe, side="right").astype(jnp.int32)
    tile_active = (g_starts < total).astype(jnp.int32)

    pos = slot.reshape(t, TOPK)
    pos1 = pos[:, 0]
    pos2 = pos[:, 1]

    xg = _sc_gather(hidden, row_token)
    eo = _ffn(xg, gate_up_w, gate_up_b, down_w, down_b, row_weight,
              tile_expert, tile_active)
    out = _sc_combine(eo, pos1, pos2, t)
    return out.reshape(batch, seq, EMB)
